# baseline (speedup 1.0000x reference)
import functools

import jax
import jax.numpy as jnp
from jax import lax
from jax.experimental import pallas as pl
from jax.experimental.pallas import tpu as pltpu
from jax.experimental.pallas import tpu_sc as plsc

B = 8
N = 4096
PATCH = 4
H = 1024

NC, NS, L = 2, 16, 16
NW = NC * NS
NPW = N // NW

ROWS16 = B * 3 * H * H // 16
MAXROW = ROWS16 - 1
PLANE16 = H * H // 16
NSTR = 30


def _coords_body(s00, s01, s10, s11, w1x, w1y, b1x, b1y,
                 r0_ref, r1_ref, r2_ref, r3_ref, r4_ref,
                 d2_ref, p0_ref, wr_ref, wc_ref):
    flat = (0.5 * (0.5 * s00[...] + 0.5 * s10[...])
            + 0.5 * (0.5 * s01[...] + 0.5 * s11[...]))
    X = jnp.dot(flat, w1x[...], preferred_element_type=jnp.float32) + b1x[...]
    Y = jnp.dot(flat, w1y[...], preferred_element_type=jnp.float32) + b1y[...]
    sx = 1.0 / (1.0 + jnp.exp(-X))
    sy = 1.0 / (1.0 + jnp.exp(-Y))
    xs = sx * float(H - 1 - PATCH) + float(PATCH // 2)
    ys = sy * float(H - 1 - PATCH) + float(PATCH // 2)
    r0f = jnp.floor(xs)
    c0f = jnp.floor(ys)
    wr_ref[...] = xs - r0f
    wc_ref[...] = ys - c0f
    h0 = r0f.astype(jnp.int32) - 2
    w0 = c0f.astype(jnp.int32) - 2
    bidx = lax.broadcasted_iota(jnp.int32, xs.shape, 0)
    base = bidx * (3 * PLANE16) + (w0 >> 7) * 64 + ((w0 & 127) >> 4)
    for q, ref in enumerate((r0_ref, r1_ref, r2_ref, r3_ref, r4_ref)):
        h = h0 + q
        ref[...] = base + (h >> 3) * 512 + (h & 7) * 8
    d2_ref[...] = jnp.where((w0 & 127) < 124, 1, 57)
    p0_ref[...] = w0 & 15


def _sc_body(tbl, w2f, rq_hbm, d2_hbm, p0_hbm, wr_hbm, wc_hbm, out_hbm,
             w2_v, idx_v, r_v, rq_v, d2_v, p0_v, wr_v, wc_v, res_v, sem):
    wid = lax.axis_index("s") * NC + lax.axis_index("c")
    n0 = wid * NPW
    pltpu.sync_copy(w2f.at[pl.ds(n0 * 96, NPW * 96)], w2_v)

    lane = lax.iota(jnp.int32, L)
    cu, cq = [], []
    for v in range(3):
        k = lane + 16 * v
        kpi = k // 12
        kpj = (k % 12) // 3
        kc = k % 3
        cu.append([kpj + b_ for b_ in (0, 1)])
        cq.append([(kc * 5 + kpi + a_) * 128 for a_ in (0, 1)])

    def batch_body(b, carry):
        off = b * N + n0
        for q in range(5):
            pltpu.sync_copy(rq_hbm.at[pl.ds(q * B * N + off, NPW)],
                            rq_v.at[pl.ds(q * NPW, NPW)])
        pltpu.sync_copy(d2_hbm.at[pl.ds(off, NPW)], d2_v)
        pltpu.sync_copy(p0_hbm.at[pl.ds(off, NPW)], p0_v)
        pltpu.sync_copy(wr_hbm.at[pl.ds(off, NPW)], wr_v)
        pltpu.sync_copy(wc_hbm.at[pl.ds(off, NPW)], wc_v)

        for c in range(3):
            for q in range(5):
                s1 = c * 5 + q
                for g in range(NPW // L):
                    row1 = rq_v[pl.ds(q * NPW + g * L, L)] + c * PLANE16
                    row2 = jnp.minimum(row1 + d2_v[pl.ds(g * L, L)], MAXROW)
                    idx_v[pl.ds(s1 * NPW + g * L, L)] = row1
                    idx_v[pl.ds((15 + s1) * NPW + g * L, L)] = row2
        copies = [
            pltpu.async_copy(tbl.at[idx_v.at[pl.ds(s_ * NPW, NPW)]],
                             r_v.at[pl.ds(s_ * NPW, NPW)], sem)
            for s_ in range(NSTR)
        ]
        for cp in copies:
            cp.wait()

        def patch_body(i, acc):
            acc0, acc1 = acc
            ivec = lax.broadcast_in_dim(i, (L,), ())
            p_i = plsc.load_gather(p0_v, [ivec])
            wr_i = plsc.load_gather(wr_v, [ivec])
            wc_i = plsc.load_gather(wc_v, [ivec])
            w00 = (1.0 - wr_i) * (1.0 - wc_i)
            w01 = (1.0 - wr_i) * wc_i
            w10 = wr_i * (1.0 - wc_i)
            w11 = wr_i * wc_i
            for v in range(3):
                taps = []
                for a_ in (0, 1):
                    for b_ in (0, 1):
                        u = cu[v][b_] + p_i
                        rrow = (cq[v][a_] + ivec
                                + jnp.where(u >= 16, 15 * 128, 0))
                        taps.append(plsc.load_gather(r_v, [rrow, u & 15]))
                pv = (w00 * taps[0] + w01 * taps[1]
                      + w10 * taps[2] + w11 * taps[3])
                wv0 = w2_v[pl.ds(i * 96 + 16 * v, L)]
                wv1 = w2_v[pl.ds(i * 96 + 48 + 16 * v, L)]
                acc0 = acc0 + pv * wv0
                acc1 = acc1 + pv * wv1
            return acc0, acc1

        acc0 = jnp.zeros((L,), jnp.float32)
        acc1 = jnp.zeros((L,), jnp.float32)
        acc0, acc1 = lax.fori_loop(0, NPW, patch_body, (acc0, acc1),
                                   unroll=4)
        t0 = plsc.cumsum(acc0)
        t1 = plsc.cumsum(acc1)
        m_last = lane == (L - 1)
        plsc.store_scatter(res_v, [2 * lax.broadcast_in_dim(b, (L,), ())],
                           t0, mask=m_last)
        plsc.store_scatter(res_v, [2 * lax.broadcast_in_dim(b, (L,), ()) + 1],
                           t1, mask=m_last)
        return carry

    lax.fori_loop(0, B, batch_body, 0)
    pltpu.sync_copy(res_v, out_hbm.at[wid])


_sc_call = functools.partial(
    pl.kernel,
    mesh=plsc.VectorSubcoreMesh(core_axis_name="c", subcore_axis_name="s"),
    compiler_params=pltpu.CompilerParams(use_tc_tiling_on_sc=False,
                                         needs_layout_passes=False),
    out_type=jax.ShapeDtypeStruct((NW, 16), jnp.float32),
    scratch_types=[
        pltpu.VMEM((NPW * 96,), jnp.float32),
        pltpu.VMEM((NSTR * NPW,), jnp.int32),
        pltpu.VMEM((NSTR * NPW, L), jnp.float32),
        pltpu.VMEM((5 * NPW,), jnp.int32),
        pltpu.VMEM((NPW,), jnp.int32),
        pltpu.VMEM((NPW,), jnp.int32),
        pltpu.VMEM((NPW,), jnp.float32),
        pltpu.VMEM((NPW,), jnp.float32),
        pltpu.VMEM((16,), jnp.float32),
        pltpu.SemaphoreType.DMA,
    ],
)(_sc_body)


def kernel(topview_image_tensor, search_views, W1, b1, W2, b2):
    top = topview_image_tensor
    s00 = top[:, :, 27::56, 27::56].reshape(B, 48)
    s01 = top[:, :, 27::56, 28::56].reshape(B, 48)
    s10 = top[:, :, 28::56, 27::56].reshape(B, 48)
    s11 = top[:, :, 28::56, 28::56].reshape(B, 48)
    w1x = W1[:, 0::2]
    w1y = W1[:, 1::2]
    b1x = b1[0::2].reshape(1, N)
    b1y = b1[1::2].reshape(1, N)

    outs = pl.pallas_call(
        _coords_body,
        out_shape=[jax.ShapeDtypeStruct((B, N), jnp.int32)] * 7
        + [jax.ShapeDtypeStruct((B, N), jnp.float32)] * 2,
    )(s00, s01, s10, s11, w1x, w1y, b1x, b1y)
    rq = jnp.concatenate([o.reshape(-1) for o in outs[:5]], axis=0)
    d2, p0 = outs[5].reshape(-1), outs[6].reshape(-1)
    wr, wc = outs[7].reshape(-1), outs[8].reshape(-1)

    svp = search_views.transpose(0, 3, 1, 2)
    tbl = (svp.reshape(B, 3, H // 8, 8, H // 128, 128)
           .transpose(0, 1, 2, 4, 3, 5).reshape(ROWS16, 16))
    w2f = W2.reshape(N, 48, 2).transpose(0, 2, 1).reshape(-1)

    partials = _sc_call(tbl, w2f, rq, d2, p0, wr, wc)
    out = partials.reshape(NW, B, 2).sum(axis=0) + b2[None, :]
    return out


# speedup vs baseline: 1.2409x; 1.2409x over previous
import functools

import jax
import jax.numpy as jnp
from jax import lax
from jax.experimental import pallas as pl
from jax.experimental.pallas import tpu as pltpu
from jax.experimental.pallas import tpu_sc as plsc

B = 8
N = 4096
PATCH = 4
H = 1024

NC, NS, L = 2, 16, 16
NW = NC * NS
NPW = N // NW

ROWS16 = B * 3 * H * H // 16
MAXROW = ROWS16 - 1
PLANE16 = H * H // 16
NSTR = 30


def _coords_body(s00, s01, s10, s11, w1x, w1y, b1x, b1y,
                 r0_ref, r1_ref, r2_ref, r3_ref, r4_ref,
                 d2_ref, p0_ref, wr_ref, wc_ref):
    flat = (0.5 * (0.5 * s00[...] + 0.5 * s10[...])
            + 0.5 * (0.5 * s01[...] + 0.5 * s11[...]))
    X = jnp.dot(flat, w1x[...], preferred_element_type=jnp.float32) + b1x[...]
    Y = jnp.dot(flat, w1y[...], preferred_element_type=jnp.float32) + b1y[...]
    sx = 1.0 / (1.0 + jnp.exp(-X))
    sy = 1.0 / (1.0 + jnp.exp(-Y))
    xs = sx * float(H - 1 - PATCH) + float(PATCH // 2)
    ys = sy * float(H - 1 - PATCH) + float(PATCH // 2)
    r0f = jnp.floor(xs)
    c0f = jnp.floor(ys)
    wr_ref[...] = xs - r0f
    wc_ref[...] = ys - c0f
    h0 = r0f.astype(jnp.int32) - 2
    w0 = c0f.astype(jnp.int32) - 2
    bidx = lax.broadcasted_iota(jnp.int32, xs.shape, 0)
    base = bidx * (3 * PLANE16) + (w0 >> 7) * 64 + ((w0 & 127) >> 4)
    for q, ref in enumerate((r0_ref, r1_ref, r2_ref, r3_ref, r4_ref)):
        h = h0 + q
        ref[...] = base + (h >> 3) * 512 + (h & 7) * 8
    d2_ref[...] = jnp.where((w0 & 127) < 124, 1, 57)
    p0_ref[...] = w0 & 15


NPC = NPW // 2
CHROWS = NSTR * NPC


def _sc_body(tbl, w2f, rq_hbm, d2_hbm, p0_hbm, wr_hbm, wc_hbm, out_hbm,
             w2_v, idx_v, r_v, rq_v, d2_v, p0_v, wr_v, wc_v, res_v,
             sem0, sem1):
    wid = lax.axis_index("s") * NC + lax.axis_index("c")
    n0 = wid * NPW
    pltpu.sync_copy(w2f.at[pl.ds(n0 * 96, NPW * 96)], w2_v)

    lane = lax.iota(jnp.int32, L)
    cu, cq = [], []
    for v in range(3):
        k = lane + 16 * v
        kpi = k // 12
        kpj = (k % 12) // 3
        kc = k % 3
        cu.append([kpj + b_ for b_ in (0, 1)])
        cq.append([(kc * 5 + kpi + a_) * NPC for a_ in (0, 1)])
    m_last = lane == (L - 1)

    def fire(t, sem):
        b = t >> 1
        half = t & 1
        tp = (t & 1) * CHROWS
        pb = (b & 1) * NPW

        @pl.when(half == 0)
        def _load_params():
            off = b * N + n0
            for q in range(5):
                pltpu.sync_copy(
                    rq_hbm.at[pl.ds(q * B * N + off, NPW)],
                    rq_v.at[pl.ds((b & 1) * 5 * NPW + q * NPW, NPW)])
            pltpu.sync_copy(d2_hbm.at[pl.ds(off, NPW)],
                            d2_v.at[pl.ds(pb, NPW)])
            pltpu.sync_copy(p0_hbm.at[pl.ds(off, NPW)],
                            p0_v.at[pl.ds(pb, NPW)])
            pltpu.sync_copy(wr_hbm.at[pl.ds(off, NPW)],
                            wr_v.at[pl.ds(pb, NPW)])
            pltpu.sync_copy(wc_hbm.at[pl.ds(off, NPW)],
                            wc_v.at[pl.ds(pb, NPW)])

        hoff = half * NPC
        for c in range(3):
            for q in range(5):
                s1 = c * 5 + q
                for g in range(NPC // L):
                    row1 = rq_v[pl.ds((b & 1) * 5 * NPW + q * NPW
                                      + hoff + g * L, L)] + c * PLANE16
                    row2 = jnp.minimum(
                        row1 + d2_v[pl.ds(pb + hoff + g * L, L)], MAXROW)
                    idx_v[pl.ds(tp + s1 * NPC + g * L, L)] = row1
                    idx_v[pl.ds(tp + (15 + s1) * NPC + g * L, L)] = row2
        for s_ in range(NSTR):
            pltpu.async_copy(tbl.at[idx_v.at[pl.ds(tp + s_ * NPC, NPC)]],
                             r_v.at[pl.ds(tp + s_ * NPC, NPC)], sem)

    def drain(t, sem):
        tp = (t & 1) * CHROWS
        for s_ in range(NSTR):
            pltpu.make_async_copy(
                tbl.at[idx_v.at[pl.ds(tp + s_ * NPC, NPC)]],
                r_v.at[pl.ds(tp + s_ * NPC, NPC)], sem).wait()

    def compute(t):
        b = t >> 1
        half = t & 1
        tp = (t & 1) * CHROWS
        pb = (b & 1) * NPW
        poff = pb + half * NPC

        def patch_body(i, acc):
            acc0, acc1 = acc
            ivec = lax.broadcast_in_dim(i, (L,), ())
            pvec = lax.broadcast_in_dim(poff + i, (L,), ())
            p_i = plsc.load_gather(p0_v, [pvec])
            wr_i = plsc.load_gather(wr_v, [pvec])
            wc_i = plsc.load_gather(wc_v, [pvec])
            w00 = (1.0 - wr_i) * (1.0 - wc_i)
            w01 = (1.0 - wr_i) * wc_i
            w10 = wr_i * (1.0 - wc_i)
            w11 = wr_i * wc_i
            ivp = ivec + tp
            for v in range(3):
                taps = []
                for a_ in (0, 1):
                    for b_ in (0, 1):
                        u = cu[v][b_] + p_i
                        rrow = (cq[v][a_] + ivp
                                + jnp.where(u >= 16, 15 * NPC, 0))
                        taps.append(plsc.load_gather(r_v, [rrow, u & 15]))
                pv = (w00 * taps[0] + w01 * taps[1]
                      + w10 * taps[2] + w11 * taps[3])
                woff = (half * NPC + i) * 96 + 16 * v
                wv0 = w2_v[pl.ds(woff, L)]
                wv1 = w2_v[pl.ds(woff + 48, L)]
                acc0 = acc0 + pv * wv0
                acc1 = acc1 + pv * wv1
            return acc0, acc1

        acc0 = jnp.zeros((L,), jnp.float32)
        acc1 = jnp.zeros((L,), jnp.float32)
        acc0, acc1 = lax.fori_loop(0, NPC, patch_body, (acc0, acc1))
        t0 = plsc.cumsum(acc0)
        t1 = plsc.cumsum(acc1)
        slot = lax.broadcast_in_dim(4 * b + 2 * half, (L,), ())
        plsc.store_scatter(res_v, [slot], t0, mask=m_last)
        plsc.store_scatter(res_v, [slot + 1], t1, mask=m_last)

    def pipe_body(t, carry):
        @pl.when(jnp.logical_and(t < 2 * B, (t & 1) == 0))
        def _f0():
            fire(t, sem0)

        @pl.when(jnp.logical_and(t < 2 * B, (t & 1) == 1))
        def _f1():
            fire(t, sem1)

        @pl.when(jnp.logical_and(t >= 1, ((t - 1) & 1) == 0))
        def _d0():
            drain(t - 1, sem0)

        @pl.when(jnp.logical_and(t >= 1, ((t - 1) & 1) == 1))
        def _d1():
            drain(t - 1, sem1)

        @pl.when(t >= 1)
        def _c():
            compute(t - 1)

        return carry

    lax.fori_loop(0, 2 * B + 1, pipe_body, 0)
    pltpu.sync_copy(res_v, out_hbm.at[wid])


_sc_call = functools.partial(
    pl.kernel,
    mesh=plsc.VectorSubcoreMesh(core_axis_name="c", subcore_axis_name="s"),
    compiler_params=pltpu.CompilerParams(use_tc_tiling_on_sc=False,
                                         needs_layout_passes=False),
    out_type=jax.ShapeDtypeStruct((NW, 32), jnp.float32),
    scratch_types=[
        pltpu.VMEM((NPW * 96,), jnp.float32),
        pltpu.VMEM((2 * CHROWS,), jnp.int32),
        pltpu.VMEM((2 * CHROWS, L), jnp.float32),
        pltpu.VMEM((2 * 5 * NPW,), jnp.int32),
        pltpu.VMEM((2 * NPW,), jnp.int32),
        pltpu.VMEM((2 * NPW,), jnp.int32),
        pltpu.VMEM((2 * NPW,), jnp.float32),
        pltpu.VMEM((2 * NPW,), jnp.float32),
        pltpu.VMEM((32,), jnp.float32),
        pltpu.SemaphoreType.DMA,
        pltpu.SemaphoreType.DMA,
    ],
)(_sc_body)


def kernel(topview_image_tensor, search_views, W1, b1, W2, b2):
    top = topview_image_tensor
    s00 = top[:, :, 27::56, 27::56].reshape(B, 48)
    s01 = top[:, :, 27::56, 28::56].reshape(B, 48)
    s10 = top[:, :, 28::56, 27::56].reshape(B, 48)
    s11 = top[:, :, 28::56, 28::56].reshape(B, 48)
    w1x = W1[:, 0::2]
    w1y = W1[:, 1::2]
    b1x = b1[0::2].reshape(1, N)
    b1y = b1[1::2].reshape(1, N)

    outs = pl.pallas_call(
        _coords_body,
        out_shape=[jax.ShapeDtypeStruct((B, N), jnp.int32)] * 7
        + [jax.ShapeDtypeStruct((B, N), jnp.float32)] * 2,
    )(s00, s01, s10, s11, w1x, w1y, b1x, b1y)
    rq = jnp.concatenate([o.reshape(-1) for o in outs[:5]], axis=0)
    d2, p0 = outs[5].reshape(-1), outs[6].reshape(-1)
    wr, wc = outs[7].reshape(-1), outs[8].reshape(-1)

    svp = search_views.transpose(0, 3, 1, 2)
    tbl = (svp.reshape(B, 3, H // 8, 8, H // 128, 128)
           .transpose(0, 1, 2, 4, 3, 5).reshape(ROWS16, 16))
    w2f = W2.reshape(N, 48, 2).transpose(0, 2, 1).reshape(-1)

    partials = _sc_call(tbl, w2f, rq, d2, p0, wr, wc)
    out = partials.reshape(NW, B, 2, 2).sum(axis=(0, 2)) + b2[None, :]
    return out
